# baseline (speedup 1.0000x reference)
import functools

import jax
import jax.numpy as jnp
from jax import lax
from jax.experimental import pallas as pl
from jax.experimental.pallas import tpu as pltpu
from jax.experimental.pallas import tpu_sc as plsc

N = 50000
D = 64
NG = 64
NBLK = 2000
EBLK = 8000
F32 = jnp.float32
SC_NC = 2
SC_NS = 16
SC_NW = SC_NC * SC_NS


def _gather2(table_p, table_q, idx_r, idx_s, ch):
    r = idx_r.shape[0]
    d = table_p.shape[1]
    dt = table_p.dtype
    nch = r // ch
    mesh = plsc.VectorSubcoreMesh(core_axis_name="c", subcore_axis_name="s")

    @functools.partial(
        pl.kernel, mesh=mesh,
        out_type=jax.ShapeDtypeStruct((r, 2 * d), dt),
        scratch_types=[pltpu.VMEM((ch,), jnp.int32),
                       pltpu.VMEM((ch,), jnp.int32),
                       pltpu.VMEM((ch, d), dt),
                       pltpu.VMEM((ch, d), dt),
                       pltpu.SemaphoreType.DMA],
        compiler_params=pltpu.CompilerParams(use_tc_tiling_on_sc=False),
    )
    def k(p_hbm, q_hbm, r_hbm, s_hbm, o_hbm, idxr, idxs, bufp, bufq, sem):
        wid = lax.axis_index("s") * SC_NC + lax.axis_index("c")
        nw = (nch - wid + SC_NW - 1) // SC_NW

        def step(i, carry):
            koff = (wid + i * SC_NW) * ch
            pltpu.sync_copy(r_hbm.at[pl.ds(koff, ch)], idxr)
            pltpu.sync_copy(s_hbm.at[pl.ds(koff, ch)], idxs)
            cp = pltpu.async_copy(p_hbm.at[idxr], bufp, sem)
            cq = pltpu.async_copy(q_hbm.at[idxs], bufq, sem)
            cp.wait()
            cq.wait()
            pltpu.sync_copy(bufp, o_hbm.at[pl.ds(koff, ch), pl.ds(0, d)])
            pltpu.sync_copy(bufq, o_hbm.at[pl.ds(koff, ch), pl.ds(d, d)])
            return carry

        lax.fori_loop(0, nw, step, 0)

    return k(table_p, table_q, idx_r, idx_s)


def _segsum2_sc(f1, recv1, f2, recv2, ch):
    qw = D // 4
    rpt = N // SC_NS
    mesh = plsc.VectorSubcoreMesh(core_axis_name="c", subcore_axis_name="s")
    zeros = jnp.zeros((N, qw), F32)

    @functools.partial(
        pl.kernel, mesh=mesh,
        out_type=jax.ShapeDtypeStruct((N, 2 * D), F32),
        scratch_types=[pltpu.VMEM((ch,), jnp.int32),
                       pltpu.VMEM((ch, qw), F32),
                       pltpu.VMEM_SHARED((N, qw), F32)],
        compiler_params=pltpu.CompilerParams(use_tc_tiling_on_sc=False),
    )
    def k(f1_hbm, r1_hbm, f2_hbm, r2_hbm, z_hbm, m_hbm, idx, fbuf, acc):
        c = lax.axis_index("c")
        s = lax.axis_index("s")
        for f_hbm, r_hbm, obase in ((f1_hbm, r1_hbm, 0), (f2_hbm, r2_hbm, D)):
            hh = f_hbm.shape[0]
            nch = hh // ch
            nw = (nch - s + SC_NS - 1) // SC_NS
            for q in range(2):
                colbase = (c * 2 + q) * qw
                pltpu.sync_copy(z_hbm.at[pl.ds(s * rpt, rpt)],
                                acc.at[pl.ds(s * rpt, rpt)])
                plsc.subcore_barrier()

                def step(i, carry, f_hbm=f_hbm, r_hbm=r_hbm, hh=hh,
                         colbase=colbase):
                    koff = (s + i * SC_NS) * ch
                    pltpu.sync_copy(r_hbm.at[pl.ds(koff, ch)], idx)
                    pltpu.sync_copy(
                        f_hbm.at[pl.ds(koff, ch), pl.ds(colbase, qw)], fbuf)
                    pltpu.sync_copy(fbuf, acc.at[idx], add=True)
                    pltpu.sync_copy(r_hbm.at[pl.ds(hh + koff, ch)], idx)
                    pltpu.sync_copy(
                        f_hbm.at[pl.ds(koff, ch), pl.ds(D + colbase, qw)],
                        fbuf)
                    pltpu.sync_copy(fbuf, acc.at[idx], add=True)
                    return carry

                lax.fori_loop(0, nw, step, 0)
                plsc.subcore_barrier()
                pltpu.sync_copy(
                    acc.at[pl.ds(s * rpt, rpt)],
                    m_hbm.at[pl.ds(s * rpt, rpt), pl.ds(obase + colbase, qw)])
                plsc.subcore_barrier()

    return k(f1, recv1, f2, recv2, zeros)


def _dot(a, b):
    return jnp.dot(a, b, preferred_element_type=F32)


def _embed_body(x_ref, w_ref, o_ref):
    o_ref[...] = _dot(x_ref[...], w_ref[...])


def _embed(x0, w):
    k = x0.shape[1]
    return pl.pallas_call(
        _embed_body,
        grid=(N // NBLK,),
        in_specs=[pl.BlockSpec((NBLK, k), lambda i: (i, 0)),
                  pl.BlockSpec((k, D), lambda i: (0, 0))],
        out_specs=pl.BlockSpec((NBLK, D), lambda i: (i, 0)),
        out_shape=jax.ShapeDtypeStruct((N, D), F32),
    )(x0, w)


def _invstat_body(dpq_ref, inv_ref, st_ref):
    i = pl.program_id(0)
    dpq = dpq_ref[...]
    d = dpq[:, 0:16] - dpq[:, 16:32]
    n2 = jnp.sum(d * d, axis=1, keepdims=True)
    n1 = jnp.sqrt(n2 + 1e-12)
    inv_ref[...] = jnp.concatenate([n1, n2], axis=1)

    @pl.when(i == 0)
    def _():
        st_ref[0] = 0.0
        st_ref[1] = 0.0
        st_ref[2] = 0.0

    st_ref[0] += jnp.sum(n1)
    st_ref[1] += jnp.sum(n2)
    st_ref[2] += jnp.sum(n2 * n2)


def _invstat(dpq32):
    r = dpq32.shape[0]
    return pl.pallas_call(
        _invstat_body,
        grid=(r // EBLK,),
        in_specs=[pl.BlockSpec((EBLK, 32), lambda i: (i, 0))],
        out_specs=[pl.BlockSpec((EBLK, 2), lambda i: (i, 0)),
                   pl.BlockSpec(memory_space=pltpu.SMEM)],
        out_shape=[jax.ShapeDtypeStruct((r, 2), F32),
                   jax.ShapeDtypeStruct((3,), F32)],
    )(dpq32)


def _proj_body(h_ref, w_ref, o1, o2, o3, o4):
    z = _dot(h_ref[...], w_ref[...]).astype(jnp.bfloat16)
    o1[...] = z[:, 0:64]
    o2[...] = z[:, 64:128]
    o3[...] = z[:, 128:192]
    o4[...] = z[:, 192:256]


def _project(h0, wcat):
    outs = [jax.ShapeDtypeStruct((N, D), jnp.bfloat16)] * 4
    return pl.pallas_call(
        _proj_body,
        grid=(N // NBLK,),
        in_specs=[pl.BlockSpec((NBLK, D), lambda i: (i, 0)),
                  pl.BlockSpec((D, 4 * D), lambda i: (0, 0))],
        out_specs=[pl.BlockSpec((NBLK, D), lambda i: (i, 0))] * 4,
        out_shape=outs,
    )(h0, wcat)


def _mlp_body(ga_ref, gb_ref, iva_ref, ivb_ref, wc_ref, b_ref, w2_ref, f_ref):
    def half(g, iv):
        g = g.astype(F32)
        z = g[:, 0:D] + g[:, D:2 * D] + _dot(iv, wc_ref[...]) + b_ref[...]
        z = jnp.maximum(z, 0.0)
        return jnp.maximum(_dot(z, w2_ref[...]), 0.0)

    fa = half(ga_ref[...], iva_ref[...])
    fb = half(gb_ref[...], ivb_ref[...])
    f_ref[...] = jnp.concatenate([fa, fb], axis=1)


def _mlp(g, inv2, wc, b, w2):
    r = g.shape[0]
    ng = (r // 2) // EBLK
    return pl.pallas_call(
        _mlp_body,
        grid=(ng,),
        in_specs=[pl.BlockSpec((EBLK, 2 * D), lambda i: (i, 0)),
                  pl.BlockSpec((EBLK, 2 * D), lambda i: (i + ng, 0)),
                  pl.BlockSpec((EBLK, 2), lambda i: (i, 0)),
                  pl.BlockSpec((EBLK, 2), lambda i: (i + ng, 0)),
                  pl.BlockSpec((2, D), lambda i: (0, 0)),
                  pl.BlockSpec((1, D), lambda i: (0, 0)),
                  pl.BlockSpec((D, D), lambda i: (0, 0))],
        out_specs=pl.BlockSpec((EBLK, 2 * D), lambda i: (i, 0)),
        out_shape=jax.ShapeDtypeStruct((r // 2, 2 * D), F32),
    )(g, g, inv2, inv2, wc, b, w2)


def _update_body(h_ref, m_ref, u_ref, pre_ref, st_ref):
    i = pl.program_id(0)
    u = u_ref[...]
    z = _dot(h_ref[...], u[0:64]) + _dot(m_ref[...], u[64:192])
    p = jnp.maximum(z, 0.0)
    pre_ref[...] = p

    @pl.when(i == 0)
    def _():
        st_ref[...] = jnp.zeros_like(st_ref)

    st_ref[...] += jnp.stack([jnp.sum(p, 0), jnp.sum(p * p, 0)])


def _update(h0, m128, u):
    return pl.pallas_call(
        _update_body,
        grid=(N // NBLK,),
        in_specs=[pl.BlockSpec((NBLK, D), lambda i: (i, 0)),
                  pl.BlockSpec((NBLK, 2 * D), lambda i: (i, 0)),
                  pl.BlockSpec((3 * D, D), lambda i: (0, 0))],
        out_specs=[pl.BlockSpec((NBLK, D), lambda i: (i, 0)),
                   pl.BlockSpec((2, D), lambda i: (0, 0))],
        out_shape=[jax.ShapeDtypeStruct((N, D), F32),
                   jax.ShapeDtypeStruct((2, D), F32)],
    )(h0, m128, u)


def _fin_body(h_ref, p_ref, sc_ref, sh_ref, o_ref):
    o_ref[...] = h_ref[...] + p_ref[...] * sc_ref[...] + sh_ref[...]


def _finalize(h0, pre, scale, shift):
    return pl.pallas_call(
        _fin_body,
        grid=(N // NBLK,),
        in_specs=[pl.BlockSpec((NBLK, D), lambda i: (i, 0)),
                  pl.BlockSpec((NBLK, D), lambda i: (i, 0)),
                  pl.BlockSpec((1, D), lambda i: (0, 0)),
                  pl.BlockSpec((1, D), lambda i: (0, 0))],
        out_specs=pl.BlockSpec((NBLK, D), lambda i: (i, 0)),
        out_shape=jax.ShapeDtypeStruct((N, D), F32),
    )(h0, pre, scale, shift)


def _segmean_body(h_ref, bi_ref, o_ref, acc_ref):
    i = pl.program_id(0)

    @pl.when(i == 0)
    def _():
        acc_ref[...] = jnp.zeros_like(acc_ref)

    mask = (bi_ref[...] == jax.lax.broadcasted_iota(
        jnp.int32, (NBLK, NG), 1)).astype(F32)
    hc = jnp.concatenate([h_ref[...], jnp.ones((NBLK, 1), F32)], axis=1)
    acc_ref[...] += jax.lax.dot_general(
        mask, hc, (((0,), (0,)), ((), ())), preferred_element_type=F32)

    @pl.when(i == pl.num_programs(0) - 1)
    def _():
        o_ref[...] = acc_ref[:, 0:D] / jnp.maximum(acc_ref[:, D:D + 1], 1.0)


def _segmean(h0, bi_col):
    return pl.pallas_call(
        _segmean_body,
        grid=(N // NBLK,),
        in_specs=[pl.BlockSpec((NBLK, D), lambda i: (i, 0)),
                  pl.BlockSpec((NBLK, 1), lambda i: (i, 0))],
        out_specs=pl.BlockSpec((NG, D), lambda i: (0, 0)),
        out_shape=jax.ShapeDtypeStruct((NG, D), F32),
        scratch_shapes=[pltpu.VMEM((NG, D + 1), F32)],
    )(h0, bi_col)


def kernel(x0, x1, x2, pos, adj_0_0_1, adj_0_0_2, adj_1_0, adj_2_0,
           batch_idx, params):
    p = params
    h0 = _embed(x0, p['emb0'])
    pos16 = jnp.pad(pos, ((0, 0), (0, 13)))

    blocks = []
    for adj, name in ((adj_0_0_1, '0_0_1'), (adj_0_0_2, '0_0_2')):
        recv = adj[0].astype(jnp.int32)
        src = adj[1].astype(jnp.int32)
        r = recv.shape[0]
        dpq32 = _gather2(pos16, pos16, recv, src, 1600)
        inv2, st = _invstat(dpq32)
        rn = jnp.float32(r)
        mu1 = st[0] / rn
        mu2 = st[1] / rn
        s1 = jnp.sqrt(jnp.maximum((st[1] + 1e-12 * rn) / rn - mu1 * mu1, 0.0)) + 1e-5
        s2 = jnp.sqrt(jnp.maximum(st[2] / rn - mu2 * mu2, 0.0)) + 1e-5
        mu = jnp.stack([mu1, mu2])
        sig = jnp.stack([s1, s2])
        blocks.append(dict(recv=recv, src=src, inv2=inv2, mu=mu, sig=sig,
                           name=name))

    for l in range(2):
        wcats = []
        for b in blocks:
            w1 = p['msg_%d_%s_w1' % (l, b['name'])]
            wcats.append(w1[0:64])
            wcats.append(w1[64:128])
        wcat = jnp.concatenate(wcats, axis=1)
        p1, q1, p2, q2 = _project(h0, wcat)
        fs = []
        for (P, Q), b in zip(((p1, q1), (p2, q2)), blocks):
            w1 = p['msg_%d_%s_w1' % (l, b['name'])]
            w2 = p['msg_%d_%s_w2' % (l, b['name'])]
            wc = w1[128:130] / b['sig'][:, None]
            badj = (-(b['mu'] / b['sig']) @ w1[128:130]).reshape(1, D)
            g = _gather2(P, Q, b['recv'], b['src'], 800)
            fs.append(_mlp(g, b['inv2'], wc, badj, w2))
        m128 = _segsum2_sc(fs[0], blocks[0]['recv'], fs[1], blocks[1]['recv'],
                           1600)
        pre, st = _update(h0, m128, p['upd_%d_0' % l])
        mu = st[0] / N
        var = jnp.maximum(st[1] / N - mu * mu, 0.0)
        scale = 1.0 / (jnp.sqrt(var) + 1e-5)
        shift = -mu * scale
        h0 = _finalize(h0, pre, scale.reshape(1, D), shift.reshape(1, D))

    return _segmean(h0, batch_idx.astype(jnp.int32).reshape(N, 1))


# speedup vs baseline: 1.4841x; 1.4841x over previous
import functools

import jax
import jax.numpy as jnp
from jax import lax
from jax.experimental import pallas as pl
from jax.experimental.pallas import tpu as pltpu
from jax.experimental.pallas import tpu_sc as plsc

N = 50000
D = 64
NG = 64
NBLK = 2000
EBLK = 8000
F32 = jnp.float32
SC_NC = 2
SC_NS = 16
SC_NW = SC_NC * SC_NS


def _gather2(table_p, table_q, idx_r, idx_s, ch):
    r = idx_r.shape[0]
    d = table_p.shape[1]
    nch = r // ch
    mesh = plsc.VectorSubcoreMesh(core_axis_name="c", subcore_axis_name="s")

    @functools.partial(
        pl.kernel, mesh=mesh,
        out_type=jax.ShapeDtypeStruct((r, 2 * d), F32),
        scratch_types=[pltpu.VMEM((ch,), jnp.int32),
                       pltpu.VMEM((ch,), jnp.int32),
                       pltpu.VMEM((ch, d), F32),
                       pltpu.VMEM((ch, d), F32),
                       pltpu.SemaphoreType.DMA],
        compiler_params=pltpu.CompilerParams(use_tc_tiling_on_sc=False),
    )
    def k(p_hbm, q_hbm, r_hbm, s_hbm, o_hbm, idxr, idxs, bufp, bufq, sem):
        wid = lax.axis_index("s") * SC_NC + lax.axis_index("c")
        nw = (nch - wid + SC_NW - 1) // SC_NW

        def step(i, carry):
            koff = (wid + i * SC_NW) * ch
            pltpu.sync_copy(r_hbm.at[pl.ds(koff, ch)], idxr)
            pltpu.sync_copy(s_hbm.at[pl.ds(koff, ch)], idxs)
            cp = pltpu.async_copy(p_hbm.at[idxr], bufp, sem)
            cq = pltpu.async_copy(q_hbm.at[idxs], bufq, sem)
            cp.wait()
            cq.wait()
            pltpu.sync_copy(bufp, o_hbm.at[pl.ds(koff, ch), pl.ds(0, d)])
            pltpu.sync_copy(bufq, o_hbm.at[pl.ds(koff, ch), pl.ds(d, d)])
            return carry

        lax.fori_loop(0, nw, step, 0)

    return k(table_p, table_q, idx_r, idx_s)


def _segsum2_sc(f1, recv1, f2, recv2, ch):
    qw = D // 4
    rpt = N // SC_NS
    mesh = plsc.VectorSubcoreMesh(core_axis_name="c", subcore_axis_name="s")
    zeros = jnp.zeros((N, qw), F32)

    @functools.partial(
        pl.kernel, mesh=mesh,
        out_type=jax.ShapeDtypeStruct((N, 2 * D), F32),
        scratch_types=[pltpu.VMEM((ch,), jnp.int32),
                       pltpu.VMEM((ch, qw), F32),
                       pltpu.VMEM_SHARED((N, qw), F32)],
        compiler_params=pltpu.CompilerParams(use_tc_tiling_on_sc=False),
    )
    def k(f1_hbm, r1_hbm, f2_hbm, r2_hbm, z_hbm, m_hbm, idx, fbuf, acc):
        c = lax.axis_index("c")
        s = lax.axis_index("s")
        for f_hbm, r_hbm, obase in ((f1_hbm, r1_hbm, 0), (f2_hbm, r2_hbm, D)):
            hh = f_hbm.shape[0]
            nch = hh // ch
            nw = (nch - s + SC_NS - 1) // SC_NS
            for q in range(2):
                colbase = (c * 2 + q) * qw
                pltpu.sync_copy(z_hbm.at[pl.ds(s * rpt, rpt)],
                                acc.at[pl.ds(s * rpt, rpt)])
                plsc.subcore_barrier()

                def step(i, carry, f_hbm=f_hbm, r_hbm=r_hbm, hh=hh,
                         colbase=colbase):
                    koff = (s + i * SC_NS) * ch
                    pltpu.sync_copy(r_hbm.at[pl.ds(koff, ch)], idx)
                    pltpu.sync_copy(
                        f_hbm.at[pl.ds(koff, ch), pl.ds(colbase, qw)], fbuf)
                    pltpu.sync_copy(fbuf, acc.at[idx], add=True)
                    pltpu.sync_copy(r_hbm.at[pl.ds(hh + koff, ch)], idx)
                    pltpu.sync_copy(
                        f_hbm.at[pl.ds(koff, ch), pl.ds(D + colbase, qw)],
                        fbuf)
                    pltpu.sync_copy(fbuf, acc.at[idx], add=True)
                    return carry

                lax.fori_loop(0, nw, step, 0)
                plsc.subcore_barrier()
                pltpu.sync_copy(
                    acc.at[pl.ds(s * rpt, rpt)],
                    m_hbm.at[pl.ds(s * rpt, rpt), pl.ds(obase + colbase, qw)])
                plsc.subcore_barrier()

    return k(f1, recv1, f2, recv2, zeros)


def _dot(a, b):
    return jnp.dot(a, b, preferred_element_type=F32)


def _embed_body(x_ref, w_ref, o_ref):
    o_ref[...] = _dot(x_ref[...], w_ref[...])


def _embed(x0, w):
    k = x0.shape[1]
    return pl.pallas_call(
        _embed_body,
        grid=(N // NBLK,),
        in_specs=[pl.BlockSpec((NBLK, k), lambda i: (i, 0)),
                  pl.BlockSpec((k, D), lambda i: (0, 0))],
        out_specs=pl.BlockSpec((NBLK, D), lambda i: (i, 0)),
        out_shape=jax.ShapeDtypeStruct((N, D), F32),
    )(x0, w)


def _invstat_body(dpq_ref, inv_ref, st_ref):
    i = pl.program_id(0)
    dpq = dpq_ref[...]
    d = dpq[:, 0:16] - dpq[:, 16:32]
    n2 = jnp.sum(d * d, axis=1, keepdims=True)
    n1 = jnp.sqrt(n2 + 1e-12)
    inv_ref[...] = jnp.concatenate([n1, n2], axis=1)

    @pl.when(i == 0)
    def _():
        st_ref[0] = 0.0
        st_ref[1] = 0.0
        st_ref[2] = 0.0

    st_ref[0] += jnp.sum(n1)
    st_ref[1] += jnp.sum(n2)
    st_ref[2] += jnp.sum(n2 * n2)


def _invstat(dpq32):
    r = dpq32.shape[0]
    return pl.pallas_call(
        _invstat_body,
        grid=(r // EBLK,),
        in_specs=[pl.BlockSpec((EBLK, 32), lambda i: (i, 0))],
        out_specs=[pl.BlockSpec((EBLK, 2), lambda i: (i, 0)),
                   pl.BlockSpec(memory_space=pltpu.SMEM)],
        out_shape=[jax.ShapeDtypeStruct((r, 2), F32),
                   jax.ShapeDtypeStruct((3,), F32)],
    )(dpq32)


def _proj_body(h_ref, w_ref, o1, o2, o3, o4):
    z = _dot(h_ref[...], w_ref[...])
    o1[...] = z[:, 0:64]
    o2[...] = z[:, 64:128]
    o3[...] = z[:, 128:192]
    o4[...] = z[:, 192:256]


def _project(h0, wcat):
    outs = [jax.ShapeDtypeStruct((N, D), F32)] * 4
    return pl.pallas_call(
        _proj_body,
        grid=(N // NBLK,),
        in_specs=[pl.BlockSpec((NBLK, D), lambda i: (i, 0)),
                  pl.BlockSpec((D, 4 * D), lambda i: (0, 0))],
        out_specs=[pl.BlockSpec((NBLK, D), lambda i: (i, 0))] * 4,
        out_shape=outs,
    )(h0, wcat)


def _mlp_body(ga_ref, gb_ref, iva_ref, ivb_ref, wc_ref, b_ref, w2_ref, f_ref):
    def half(g, iv):
        z = g[:, 0:D] + g[:, D:2 * D] + _dot(iv, wc_ref[...]) + b_ref[...]
        z = jnp.maximum(z, 0.0)
        return jnp.maximum(_dot(z, w2_ref[...]), 0.0)

    fa = half(ga_ref[...], iva_ref[...])
    fb = half(gb_ref[...], ivb_ref[...])
    f_ref[...] = jnp.concatenate([fa, fb], axis=1)


def _mlp(g, inv2, wc, b, w2):
    r = g.shape[0]
    ng = (r // 2) // EBLK
    return pl.pallas_call(
        _mlp_body,
        grid=(ng,),
        in_specs=[pl.BlockSpec((EBLK, 2 * D), lambda i: (i, 0)),
                  pl.BlockSpec((EBLK, 2 * D), lambda i: (i + ng, 0)),
                  pl.BlockSpec((EBLK, 2), lambda i: (i, 0)),
                  pl.BlockSpec((EBLK, 2), lambda i: (i + ng, 0)),
                  pl.BlockSpec((2, D), lambda i: (0, 0)),
                  pl.BlockSpec((1, D), lambda i: (0, 0)),
                  pl.BlockSpec((D, D), lambda i: (0, 0))],
        out_specs=pl.BlockSpec((EBLK, 2 * D), lambda i: (i, 0)),
        out_shape=jax.ShapeDtypeStruct((r // 2, 2 * D), F32),
    )(g, g, inv2, inv2, wc, b, w2)


def _update_body(h_ref, m_ref, u_ref, pre_ref, st_ref):
    i = pl.program_id(0)
    u = u_ref[...]
    z = _dot(h_ref[...], u[0:64]) + _dot(m_ref[...], u[64:192])
    p = jnp.maximum(z, 0.0)
    pre_ref[...] = p

    @pl.when(i == 0)
    def _():
        st_ref[...] = jnp.zeros_like(st_ref)

    st_ref[...] += jnp.stack([jnp.sum(p, 0), jnp.sum(p * p, 0)])


def _update(h0, m128, u):
    return pl.pallas_call(
        _update_body,
        grid=(N // NBLK,),
        in_specs=[pl.BlockSpec((NBLK, D), lambda i: (i, 0)),
                  pl.BlockSpec((NBLK, 2 * D), lambda i: (i, 0)),
                  pl.BlockSpec((3 * D, D), lambda i: (0, 0))],
        out_specs=[pl.BlockSpec((NBLK, D), lambda i: (i, 0)),
                   pl.BlockSpec((2, D), lambda i: (0, 0))],
        out_shape=[jax.ShapeDtypeStruct((N, D), F32),
                   jax.ShapeDtypeStruct((2, D), F32)],
    )(h0, m128, u)


def _fin_body(h_ref, p_ref, sc_ref, sh_ref, o_ref):
    o_ref[...] = h_ref[...] + p_ref[...] * sc_ref[...] + sh_ref[...]


def _finalize(h0, pre, scale, shift):
    return pl.pallas_call(
        _fin_body,
        grid=(N // NBLK,),
        in_specs=[pl.BlockSpec((NBLK, D), lambda i: (i, 0)),
                  pl.BlockSpec((NBLK, D), lambda i: (i, 0)),
                  pl.BlockSpec((1, D), lambda i: (0, 0)),
                  pl.BlockSpec((1, D), lambda i: (0, 0))],
        out_specs=pl.BlockSpec((NBLK, D), lambda i: (i, 0)),
        out_shape=jax.ShapeDtypeStruct((N, D), F32),
    )(h0, pre, scale, shift)


def _segmean_body(h_ref, bi_ref, o_ref, acc_ref):
    i = pl.program_id(0)

    @pl.when(i == 0)
    def _():
        acc_ref[...] = jnp.zeros_like(acc_ref)

    mask = (bi_ref[...] == jax.lax.broadcasted_iota(
        jnp.int32, (NBLK, NG), 1)).astype(F32)
    hc = jnp.concatenate([h_ref[...], jnp.ones((NBLK, 1), F32)], axis=1)
    acc_ref[...] += jax.lax.dot_general(
        mask, hc, (((0,), (0,)), ((), ())), preferred_element_type=F32)

    @pl.when(i == pl.num_programs(0) - 1)
    def _():
        o_ref[...] = acc_ref[:, 0:D] / jnp.maximum(acc_ref[:, D:D + 1], 1.0)


def _segmean(h0, bi_col):
    return pl.pallas_call(
        _segmean_body,
        grid=(N // NBLK,),
        in_specs=[pl.BlockSpec((NBLK, D), lambda i: (i, 0)),
                  pl.BlockSpec((NBLK, 1), lambda i: (i, 0))],
        out_specs=pl.BlockSpec((NG, D), lambda i: (0, 0)),
        out_shape=jax.ShapeDtypeStruct((NG, D), F32),
        scratch_shapes=[pltpu.VMEM((NG, D + 1), F32)],
    )(h0, bi_col)


def kernel(x0, x1, x2, pos, adj_0_0_1, adj_0_0_2, adj_1_0, adj_2_0,
           batch_idx, params):
    p = params
    h0 = _embed(x0, p['emb0'])
    pos16 = jnp.pad(pos, ((0, 0), (0, 13)))

    blocks = []
    for adj, name in ((adj_0_0_1, '0_0_1'), (adj_0_0_2, '0_0_2')):
        recv = adj[0].astype(jnp.int32)
        src = adj[1].astype(jnp.int32)
        r = recv.shape[0]
        dpq32 = _gather2(pos16, pos16, recv, src, 1600)
        inv2, st = _invstat(dpq32)
        rn = jnp.float32(r)
        mu1 = st[0] / rn
        mu2 = st[1] / rn
        s1 = jnp.sqrt(jnp.maximum((st[1] + 1e-12 * rn) / rn - mu1 * mu1, 0.0)) + 1e-5
        s2 = jnp.sqrt(jnp.maximum(st[2] / rn - mu2 * mu2, 0.0)) + 1e-5
        mu = jnp.stack([mu1, mu2])
        sig = jnp.stack([s1, s2])
        blocks.append(dict(recv=recv, src=src, inv2=inv2, mu=mu, sig=sig,
                           name=name))

    for l in range(2):
        wcats = []
        for b in blocks:
            w1 = p['msg_%d_%s_w1' % (l, b['name'])]
            wcats.append(w1[0:64])
            wcats.append(w1[64:128])
        wcat = jnp.concatenate(wcats, axis=1)
        p1, q1, p2, q2 = _project(h0, wcat)
        fs = []
        for (P, Q), b in zip(((p1, q1), (p2, q2)), blocks):
            w1 = p['msg_%d_%s_w1' % (l, b['name'])]
            w2 = p['msg_%d_%s_w2' % (l, b['name'])]
            wc = w1[128:130] / b['sig'][:, None]
            badj = (-(b['mu'] / b['sig']) @ w1[128:130]).reshape(1, D)
            g = _gather2(P, Q, b['recv'], b['src'], 800)
            fs.append(_mlp(g, b['inv2'], wc, badj, w2))
        m128 = _segsum2_sc(fs[0], blocks[0]['recv'], fs[1], blocks[1]['recv'],
                           1600)
        pre, st = _update(h0, m128, p['upd_%d_0' % l])
        mu = st[0] / N
        var = jnp.maximum(st[1] / N - mu * mu, 0.0)
        scale = 1.0 / (jnp.sqrt(var) + 1e-5)
        shift = -mu * scale
        h0 = _finalize(h0, pre, scale.reshape(1, D), shift.reshape(1, D))

    return _segmean(h0, batch_idx.astype(jnp.int32).reshape(N, 1))
